# baseline (speedup 1.0000x reference)
import jax
import jax.numpy as jnp
from jax.experimental import pallas as pl
from jax.experimental.pallas import tpu as pltpu

_BLOCK_ROWS = 1024


def _body(x_ref, o_ref):
    i = pl.program_id(0)
    base = i * _BLOCK_ROWS
    row_ids = (jax.lax.broadcasted_iota(jnp.int32, x_ref.shape, 0) + base
               ).astype(jnp.float32)
    o_ref[...] = x_ref[...] * 2.0 + row_ids


def kernel(input_tensor):
    n, d = input_tensor.shape
    grid = (n // _BLOCK_ROWS,)
    return pl.pallas_call(
        _body,
        grid=grid,
        in_specs=[pl.BlockSpec((_BLOCK_ROWS, d), lambda i: (i, 0))],
        out_specs=pl.BlockSpec((_BLOCK_ROWS, d), lambda i: (i, 0)),
        out_shape=jax.ShapeDtypeStruct((n, d), input_tensor.dtype),
        compiler_params=pltpu.CompilerParams(
            dimension_semantics=("parallel",),
        ),
    )(input_tensor)


# speedup vs baseline: 1.0346x; 1.0346x over previous
import jax
import jax.numpy as jnp
from jax.experimental import pallas as pl
from jax.experimental.pallas import tpu as pltpu

_BLOCK_ROWS = 2048


def _body(x_ref, o_ref):
    i = pl.program_id(0)
    base = i * _BLOCK_ROWS
    row_ids = (jax.lax.broadcasted_iota(jnp.int32, x_ref.shape, 0) + base
               ).astype(jnp.float32)
    o_ref[...] = x_ref[...] * 2.0 + row_ids


def kernel(input_tensor):
    n, d = input_tensor.shape
    grid = (n // _BLOCK_ROWS,)
    return pl.pallas_call(
        _body,
        grid=grid,
        in_specs=[pl.BlockSpec((_BLOCK_ROWS, d), lambda i: (i, 0))],
        out_specs=pl.BlockSpec((_BLOCK_ROWS, d), lambda i: (i, 0)),
        out_shape=jax.ShapeDtypeStruct((n, d), input_tensor.dtype),
        compiler_params=pltpu.CompilerParams(
            dimension_semantics=("parallel",),
        ),
    )(input_tensor)


# speedup vs baseline: 1.0360x; 1.0014x over previous
import jax
import jax.numpy as jnp
from jax.experimental import pallas as pl
from jax.experimental.pallas import tpu as pltpu

_BLOCK_ROWS = 2048


def _body(x_ref, o_ref):
    i = pl.program_id(0)
    base = i * _BLOCK_ROWS
    rows = x_ref.shape[0]
    row_col = (jax.lax.broadcasted_iota(jnp.int32, (rows, 1), 0) + base
               ).astype(jnp.float32)
    o_ref[...] = x_ref[...] * 2.0 + row_col


def kernel(input_tensor):
    n, d = input_tensor.shape
    grid = (n // _BLOCK_ROWS,)
    return pl.pallas_call(
        _body,
        grid=grid,
        in_specs=[pl.BlockSpec((_BLOCK_ROWS, d), lambda i: (i, 0))],
        out_specs=pl.BlockSpec((_BLOCK_ROWS, d), lambda i: (i, 0)),
        out_shape=jax.ShapeDtypeStruct((n, d), input_tensor.dtype),
        compiler_params=pltpu.CompilerParams(
            dimension_semantics=("parallel",),
        ),
    )(input_tensor)
